# baseline (speedup 1.0000x reference)
import jax
import jax.numpy as jnp
from jax import lax
from jax.experimental import pallas as pl
from jax.experimental.pallas import tpu as pltpu
from jax.experimental.pallas import tpu_sc as plsc

NODE_SIZE = 1000000
EMBED_DIM = 32
BATCH = 16384

NUM_CORES = 2
NUM_SUBCORES = 16
LANES = 16
NUM_WORKERS = NUM_CORES * NUM_SUBCORES
B_PER_W = BATCH // NUM_WORKERS
NUM_VECS = B_PER_W // LANES


def _rsqrt_newton(x):
  y = plsc.bitcast(
      jnp.int32(0x5F3759DF) - lax.shift_right_logical(
          plsc.bitcast(x, jnp.int32), jnp.int32(1)),
      jnp.float32)
  half_x = x * 0.5
  for _ in range(3):
    y = y * (1.5 - half_x * y * y)
  return y


HALF = B_PER_W // 2
HALF_VECS = HALF // LANES


def _body(node_hbm, t32_hbm, out_hbm, idx_v, rows8_v, out_v, sem):
  wid = lax.axis_index("s") * NUM_CORES + lax.axis_index("c")
  base = wid * B_PER_W
  pltpu.sync_copy(node_hbm.at[pl.ds(base, B_PER_W)], idx_v)

  lane_iota = lax.iota(jnp.int32, LANES)

  for h in range(2):
    def fire_vec(c, carry):
      vec = idx_v[pl.ds(h * HALF + c * LANES, LANES)]
      for j in range(LANES):
        r = lax.squeeze(lax.slice(vec, [j], [j + 1]), [0])
        start = pl.multiple_of((r >> 3) * 8, 8)
        pltpu.async_copy(t32_hbm.at[:, pl.ds(start, 8)],
                         rows8_v.at[:, pl.ds(8 * (c * LANES + j), 8)], sem)
      return carry

    lax.fori_loop(0, HALF_VECS, fire_vec, 0)
    pltpu.make_async_copy(
        t32_hbm.at[:, pl.ds(0, 8 * HALF)], rows8_v, sem).wait()

    def norm_group(g, carry):
      idx_g = idx_v[pl.ds(h * HALF + g * LANES, LANES)]
      col8 = (g * LANES + lane_iota) * 8 + (idx_g & 7)
      vals = [
          plsc.load_gather(rows8_v, [jnp.full((LANES,), d, jnp.int32), col8])
          for d in range(EMBED_DIM)]
      s = vals[0] * vals[0]
      for d in range(1, EMBED_DIM):
        s = s + vals[d] * vals[d]
      scale = _rsqrt_newton(jnp.maximum(s, 1e-24))
      cols = pl.ds(h * HALF + g * LANES, LANES)
      for d in range(EMBED_DIM):
        out_v[d, cols] = vals[d] * scale
      return carry

    lax.fori_loop(0, HALF_VECS, norm_group, 0)

  pltpu.sync_copy(out_v, out_hbm.at[:, pl.ds(base, B_PER_W)])


@jax.jit
def _lookup_normalize(node, table):
  t32 = table.T
  mesh = plsc.VectorSubcoreMesh(
      core_axis_name="c", subcore_axis_name="s",
      num_cores=NUM_CORES, num_subcores=NUM_SUBCORES)
  out_cm = pl.kernel(
      _body,
      out_type=jax.ShapeDtypeStruct((EMBED_DIM, BATCH), jnp.float32),
      mesh=mesh,
      scratch_types=[
          pltpu.VMEM((B_PER_W,), jnp.int32),
          pltpu.VMEM((EMBED_DIM, 8 * (B_PER_W // 2)), jnp.float32),
          pltpu.VMEM((EMBED_DIM, B_PER_W), jnp.float32),
          pltpu.SemaphoreType.DMA,
      ],
      compiler_params=pltpu.CompilerParams(
          needs_layout_passes=False, use_tc_tiling_on_sc=False),
  )(node, t32)
  return out_cm.T


def kernel(node, table):
  return _lookup_normalize(node.astype(jnp.int32), table)


# speedup vs baseline: 5.0396x; 5.0396x over previous
import jax
import jax.numpy as jnp
from jax import lax
from jax.experimental import pallas as pl
from jax.experimental.pallas import tpu as pltpu
from jax.experimental.pallas import tpu_sc as plsc

NODE_SIZE = 1000000
EMBED_DIM = 32
BATCH = 16384

NUM_CORES = 2
NUM_SUBCORES = 16
LANES = 16
NUM_WORKERS = NUM_CORES * NUM_SUBCORES
B_PER_W = BATCH // NUM_WORKERS
NUM_VECS = B_PER_W // LANES

PACK = 128 // EMBED_DIM
PACKED_ROWS = NODE_SIZE // PACK


def _rsqrt_newton(x):
  y = plsc.bitcast(
      jnp.int32(0x5F3759DF) - lax.shift_right_logical(
          plsc.bitcast(x, jnp.int32), jnp.int32(1)),
      jnp.float32)
  half_x = x * 0.5
  for _ in range(3):
    y = y * (1.5 - half_x * y * y)
  return y


def _body(node_hbm, tr_hbm, out_hbm, idx_v, qidx_v, rows_v, out_v, sem):
  wid = lax.axis_index("s") * NUM_CORES + lax.axis_index("c")
  base = wid * B_PER_W
  pltpu.sync_copy(node_hbm.at[pl.ds(base, B_PER_W)], idx_v)

  def qidx_vec(g, carry):
    sl = pl.ds(g * LANES, LANES)
    qidx_v[sl] = lax.shift_right_logical(idx_v[sl], 2)
    return carry

  lax.fori_loop(0, NUM_VECS, qidx_vec, 0)
  pltpu.async_copy(tr_hbm.at[qidx_v], rows_v, sem).wait()

  lane_iota = lax.iota(jnp.int32, LANES)

  def norm_group(g, carry):
    sl = pl.ds(g * LANES, LANES)
    n_vec = g * LANES + lane_iota
    colbase = (idx_v[sl] & (PACK - 1)) * EMBED_DIM
    vals = [plsc.load_gather(rows_v, [n_vec, colbase + d])
            for d in range(EMBED_DIM)]
    s = vals[0] * vals[0]
    for d in range(1, EMBED_DIM):
      s = s + vals[d] * vals[d]
    scale = _rsqrt_newton(jnp.maximum(s, 1e-24))
    for d in range(EMBED_DIM):
      out_v[d, sl] = vals[d] * scale
    return carry

  lax.fori_loop(0, NUM_VECS, norm_group, 0)
  pltpu.sync_copy(out_v, out_hbm.at[:, pl.ds(base, B_PER_W)])


@jax.jit
def _lookup_normalize(node, table):
  tr = table.reshape(PACKED_ROWS, 128)
  mesh = plsc.VectorSubcoreMesh(
      core_axis_name="c", subcore_axis_name="s",
      num_cores=NUM_CORES, num_subcores=NUM_SUBCORES)
  out_cm = pl.kernel(
      _body,
      out_type=jax.ShapeDtypeStruct((EMBED_DIM, BATCH), jnp.float32),
      mesh=mesh,
      scratch_types=[
          pltpu.VMEM((B_PER_W,), jnp.int32),
          pltpu.VMEM((B_PER_W,), jnp.int32),
          pltpu.VMEM((B_PER_W, 128), jnp.float32),
          pltpu.VMEM((EMBED_DIM, B_PER_W), jnp.float32),
          pltpu.SemaphoreType.DMA,
      ],
      compiler_params=pltpu.CompilerParams(needs_layout_passes=False),
  )(node, tr)
  return out_cm.T


def kernel(node, table):
  return _lookup_normalize(node.astype(jnp.int32), table)


# speedup vs baseline: 5.1753x; 1.0269x over previous
import jax
import jax.numpy as jnp
from jax import lax
from jax.experimental import pallas as pl
from jax.experimental.pallas import tpu as pltpu
from jax.experimental.pallas import tpu_sc as plsc

NODE_SIZE = 1000000
EMBED_DIM = 32
BATCH = 16384

NUM_CORES = 2
NUM_SUBCORES = 16
LANES = 16
NUM_WORKERS = NUM_CORES * NUM_SUBCORES
B_PER_W = BATCH // NUM_WORKERS
NUM_VECS = B_PER_W // LANES

def _rsqrt_newton(x):
  y = plsc.bitcast(
      jnp.int32(0x5F3759DF) - lax.shift_right_logical(
          plsc.bitcast(x, jnp.int32), jnp.int32(1)),
      jnp.float32)
  half_x = x * 0.5
  for _ in range(3):
    y = y * (1.5 - half_x * y * y)
  return y


def _body(node_hbm, tr_hbm, out_hbm, idx_v, rows_v, out_v, sem):
  wid = lax.axis_index("s") * NUM_CORES + lax.axis_index("c")
  base = wid * B_PER_W
  pltpu.sync_copy(node_hbm.at[pl.ds(base, B_PER_W)], idx_v)
  pltpu.async_copy(tr_hbm.at[idx_v], rows_v, sem).wait()

  lane_iota = lax.iota(jnp.int32, LANES)

  def norm_group(g, carry):
    sl = pl.ds(g * LANES, LANES)
    n_vec = g * LANES + lane_iota
    vals = [plsc.load_gather(rows_v, [n_vec, jnp.full((LANES,), d, jnp.int32)])
            for d in range(EMBED_DIM)]
    s = vals[0] * vals[0]
    for d in range(1, EMBED_DIM):
      s = s + vals[d] * vals[d]
    scale = _rsqrt_newton(jnp.maximum(s, 1e-24))
    for d in range(EMBED_DIM):
      out_v[d, sl] = vals[d] * scale
    return carry

  lax.fori_loop(0, NUM_VECS, norm_group, 0)
  pltpu.sync_copy(out_v, out_hbm.at[:, pl.ds(base, B_PER_W)])


@jax.jit
def _lookup_normalize(node, table):
  tr = jnp.pad(table, ((0, 0), (0, 128 - EMBED_DIM)))
  mesh = plsc.VectorSubcoreMesh(
      core_axis_name="c", subcore_axis_name="s",
      num_cores=NUM_CORES, num_subcores=NUM_SUBCORES)
  out_cm = pl.kernel(
      _body,
      out_type=jax.ShapeDtypeStruct((EMBED_DIM, BATCH), jnp.float32),
      mesh=mesh,
      scratch_types=[
          pltpu.VMEM((B_PER_W,), jnp.int32),
          pltpu.VMEM((B_PER_W, 128), jnp.float32),
          pltpu.VMEM((EMBED_DIM, B_PER_W), jnp.float32),
          pltpu.SemaphoreType.DMA,
      ],
      compiler_params=pltpu.CompilerParams(needs_layout_passes=False),
  )(node, tr)
  return out_cm.T


def kernel(node, table):
  return _lookup_normalize(node.astype(jnp.int32), table)


# speedup vs baseline: 8.3608x; 1.6155x over previous
import jax
import jax.numpy as jnp
from jax import lax
from jax.experimental import pallas as pl
from jax.experimental.pallas import tpu as pltpu
from jax.experimental.pallas import tpu_sc as plsc

NODE_SIZE = 1000000
EMBED_DIM = 32
BATCH = 16384

NUM_CORES = 2
NUM_SUBCORES = 16
LANES = 16
NUM_WORKERS = NUM_CORES * NUM_SUBCORES
B_PER_W = BATCH // NUM_WORKERS
NUM_VECS = B_PER_W // LANES


def _rsqrt_newton(x):
  y = plsc.bitcast(
      jnp.int32(0x5F3759DF) - lax.shift_right_logical(
          plsc.bitcast(x, jnp.int32), jnp.int32(1)),
      jnp.float32)
  half_x = x * 0.5
  for _ in range(3):
    y = y * (1.5 - half_x * y * y)
  return y


def _body(node_hbm, table_hbm, out_hbm, idx_v, rows_v, out_v, sem):
  wid = lax.axis_index("s") * NUM_CORES + lax.axis_index("c")
  base = wid * B_PER_W
  pltpu.sync_copy(node_hbm.at[pl.ds(base, B_PER_W)], idx_v)

  def fire_vec(c, carry):
    vec = idx_v[pl.ds(c * LANES, LANES)]
    for j in range(LANES):
      r = lax.squeeze(lax.slice(vec, [j], [j + 1]), [0])
      pltpu.async_copy(table_hbm.at[r], rows_v.at[c * LANES + j], sem)
    return carry

  lax.fori_loop(0, NUM_VECS, fire_vec, 0)
  pltpu.make_async_copy(
      table_hbm.at[pl.ds(0, B_PER_W)], rows_v, sem).wait()

  lane_iota = lax.iota(jnp.int32, LANES)

  def norm_group(g, carry):
    sl = pl.ds(g * LANES, LANES)
    n_vec = g * LANES + lane_iota
    vals = [plsc.load_gather(rows_v, [n_vec, jnp.full((LANES,), d, jnp.int32)])
            for d in range(EMBED_DIM)]
    s = vals[0] * vals[0]
    for d in range(1, EMBED_DIM):
      s = s + vals[d] * vals[d]
    scale = _rsqrt_newton(jnp.maximum(s, 1e-24))
    for d in range(EMBED_DIM):
      out_v[d, sl] = vals[d] * scale
    return carry

  lax.fori_loop(0, NUM_VECS, norm_group, 0)
  pltpu.sync_copy(out_v, out_hbm.at[:, pl.ds(base, B_PER_W)])


@jax.jit
def _lookup_normalize(node, table):
  mesh = plsc.VectorSubcoreMesh(
      core_axis_name="c", subcore_axis_name="s",
      num_cores=NUM_CORES, num_subcores=NUM_SUBCORES)
  out_cm = pl.kernel(
      _body,
      out_type=jax.ShapeDtypeStruct((EMBED_DIM, BATCH), jnp.float32),
      mesh=mesh,
      scratch_types=[
          pltpu.VMEM((B_PER_W,), jnp.int32),
          pltpu.VMEM((B_PER_W, EMBED_DIM), jnp.float32),
          pltpu.VMEM((EMBED_DIM, B_PER_W), jnp.float32),
          pltpu.SemaphoreType.DMA,
      ],
      compiler_params=pltpu.CompilerParams(needs_layout_passes=False),
  )(node, table)
  return out_cm.T


def kernel(node, table):
  return _lookup_normalize(node.astype(jnp.int32), table)
